# baseline (speedup 1.0000x reference)
import functools

import jax
import jax.numpy as jnp
from jax import lax
from jax.experimental import pallas as pl
from jax.experimental.pallas import tpu as pltpu
from jax.experimental.pallas import tpu_sc as plsc

_N = 100000
_C = 3
_NP = 100096
_K = 32
_NSUB = 32
_W = 3200
_LAST = _NP - (_NSUB - 1) * _W
_TRUE_WORD = 0x00010101
_LANES = 16


def _sc_body(idx_hbm, out_hbm, zbuf, idxv):
    wid = lax.axis_index("s") * 2 + lax.axis_index("c")
    pltpu.sync_copy(idx_hbm, idxv)
    for j in range(_W // _LANES):
        zbuf[pl.ds(j * _LANES, _LANES)] = jnp.zeros((_LANES,), jnp.int32)
    w0 = wid * _W
    ones = jnp.full((_LANES,), _TRUE_WORD, jnp.int32)
    for h in range(_K // _LANES):
        v = idxv[pl.ds(h * _LANES, _LANES)]
        m = (v >= w0) & (v < w0 + _W)
        plsc.store_scatter(zbuf, [v - w0], ones, mask=m)
    @pl.when(wid < _NSUB - 1)
    def _():
        pltpu.sync_copy(zbuf, out_hbm.at[pl.ds(w0, _W)])
    @pl.when(wid == _NSUB - 1)
    def _():
        pltpu.sync_copy(zbuf.at[pl.ds(0, _LAST)],
                        out_hbm.at[pl.ds((_NSUB - 1) * _W, _LAST)])


@functools.cache
def _sc_call():
    mesh = plsc.VectorSubcoreMesh(core_axis_name="c", subcore_axis_name="s")
    return functools.partial(
        pl.kernel,
        mesh=mesh,
        compiler_params=pltpu.CompilerParams(needs_layout_passes=False),
        out_type=jax.ShapeDtypeStruct((_NP,), jnp.int32),
        scratch_types=[
            pltpu.VMEM((_W,), jnp.int32),
            pltpu.VMEM((_K,), jnp.int32),
        ],
    )(_sc_body)


def kernel(pos, idx):
    del pos
    words = _sc_call()(idx.astype(jnp.int32))
    return words[:_N, None] != jnp.zeros((1, _C), jnp.int32)


# speedup vs baseline: 1.0034x; 1.0034x over previous
import functools

import jax
import jax.numpy as jnp
from jax import lax
from jax.experimental import pallas as pl
from jax.experimental.pallas import tpu as pltpu
from jax.experimental.pallas import tpu_sc as plsc

_N = 100000
_C = 3
_NP = 100096
_K = 32
_NSUB = 32
_W = 3200
_LAST = _NP - (_NSUB - 1) * _W
_TRUE_WORD = 0x00010101
_LANES = 16


def _sc_body(idx_hbm, out_hbm, zbuf, idxv):
    wid = lax.axis_index("s") * 2 + lax.axis_index("c")
    pltpu.sync_copy(idx_hbm, idxv)
    for j in range(_W // _LANES):
        zbuf[pl.ds(j * _LANES, _LANES)] = jnp.zeros((_LANES,), jnp.int32)
    w0 = wid * _W
    ones = jnp.full((_LANES,), _TRUE_WORD, jnp.int32)
    for h in range(_K // _LANES):
        v = idxv[pl.ds(h * _LANES, _LANES)]
        m = (v >= w0) & (v < w0 + _W)
        plsc.store_scatter(zbuf, [v - w0], ones, mask=m)
    @pl.when(wid < _NSUB - 1)
    def _():
        pltpu.sync_copy(zbuf, out_hbm.at[pl.ds(w0, _W)])
    @pl.when(wid == _NSUB - 1)
    def _():
        pltpu.sync_copy(zbuf.at[pl.ds(0, _LAST)],
                        out_hbm.at[pl.ds((_NSUB - 1) * _W, _LAST)])


@functools.cache
def _sc_call():
    mesh = plsc.VectorSubcoreMesh(core_axis_name="c", subcore_axis_name="s")
    return functools.partial(
        pl.kernel,
        mesh=mesh,
        compiler_params=pltpu.CompilerParams(
            needs_layout_passes=False,
            skip_device_barrier=True,
            disable_bounds_checks=True,
            disable_semaphore_checks=True,
        ),
        out_type=jax.ShapeDtypeStruct((_NP,), jnp.int32),
        scratch_types=[
            pltpu.VMEM((_W,), jnp.int32),
            pltpu.VMEM((_K,), jnp.int32),
        ],
    )(_sc_body)


def kernel(pos, idx):
    del pos
    words = _sc_call()(idx.astype(jnp.int32))
    return words[:_N, None] != jnp.zeros((1, _C), jnp.int32)
